# Initial kernel scaffold; baseline (speedup 1.0000x reference)
import jax
import jax.numpy as jnp
from jax.experimental import pallas as pl


def kernel(x, edge_index, batch, W1, b1, W2, b2, W3, b3, Wl, bl):
    raise NotImplementedError("write your pallas kernel here")


# speedup vs baseline: 18.8096x; 18.8096x over previous
import functools

import jax
import jax.numpy as jnp
from jax import lax
from jax.experimental import pallas as pl
from jax.experimental.pallas import tpu as pltpu
from jax.experimental.pallas import tpu_sc as plsc

N = 10000
E = 320000
D = 128
G = 64

NC = 2
NS = 16
NW = NC * NS
CW = 128
WPT = 79
EPAD = WPT * CW * NW
NP = 10240
RPT = NP // NS
BLK = 512
NB = NP // BLK
DUMMY0 = N + 16
NDUM = NP - DUMMY0


@functools.lru_cache(maxsize=None)
def _sc_kernels():
    mesh = plsc.VectorSubcoreMesh(
        core_axis_name="c", subcore_axis_name="s",
        num_cores=NC, num_subcores=NS)

    @functools.partial(
        pl.kernel,
        out_type=jax.ShapeDtypeStruct((NC, NP, D), jnp.float32),
        mesh=mesh,
        scratch_types=[
            pltpu.VMEM((WPT, CW), jnp.int32),
            pltpu.VMEM((WPT, CW), jnp.int32),
            pltpu.VMEM((CW, D), jnp.float32),
            pltpu.VMEM((64, D), jnp.float32),
            pltpu.VMEM_SHARED((NP, D), jnp.float32),
            pltpu.SemaphoreType.DMA,
        ],
    )
    def agg(src_hbm, dst_hbm, hp_hbm, out_hbm, src_v, dst_v, rows_v, zb_v,
            acc_sh, sem):
        c = lax.axis_index("c")
        s = lax.axis_index("s")
        w = c * NS + s

        def zfill(i, _):
            zb_v[i // 8, pl.ds((i % 8) * 16, 16)] = jnp.zeros((16,), jnp.float32)
            return 0
        lax.fori_loop(0, 64 * 8, zfill, 0)

        def zcopy(i, _):
            pltpu.sync_copy(zb_v, acc_sh.at[pl.ds(s * RPT + i * 64, 64)])
            return 0
        lax.fori_loop(0, RPT // 64, zcopy, 0)

        pltpu.sync_copy(src_hbm.at[w], src_v)
        pltpu.sync_copy(dst_hbm.at[w], dst_v)
        plsc.subcore_barrier()

        def body(j, _):
            pltpu.async_copy(hp_hbm.at[src_v.at[j]], rows_v, sem).wait()
            pltpu.sync_copy(rows_v, acc_sh.at[dst_v.at[j]], add=True)
            return 0
        lax.fori_loop(0, WPT, body, 0)

        plsc.subcore_barrier()
        pltpu.sync_copy(acc_sh.at[pl.ds(s * RPT, RPT)],
                        out_hbm.at[c, pl.ds(s * RPT, RPT)])

    @functools.partial(
        pl.kernel,
        out_type=jax.ShapeDtypeStruct((NC, NP, 16), jnp.float32),
        mesh=mesh,
        scratch_types=[
            pltpu.VMEM((WPT, CW), jnp.int32),
            pltpu.VMEM((CW, 16), jnp.float32),
            pltpu.VMEM((64, 16), jnp.float32),
            pltpu.VMEM_SHARED((NP, 16), jnp.float32),
        ],
    )
    def deg(dst_hbm, out_hbm, dst_v, ones_v, zb_v, acc_sh):
        c = lax.axis_index("c")
        s = lax.axis_index("s")
        w = c * NS + s

        def ofill(i, _):
            ones_v[i, :] = jnp.ones((16,), jnp.float32)
            return 0
        lax.fori_loop(0, CW, ofill, 0)

        def zfill(i, _):
            zb_v[i, :] = jnp.zeros((16,), jnp.float32)
            return 0
        lax.fori_loop(0, 64, zfill, 0)

        def zcopy(i, _):
            pltpu.sync_copy(zb_v, acc_sh.at[pl.ds(s * RPT + i * 64, 64)])
            return 0
        lax.fori_loop(0, RPT // 64, zcopy, 0)

        pltpu.sync_copy(dst_hbm.at[w], dst_v)
        plsc.subcore_barrier()

        def body(j, _):
            pltpu.sync_copy(ones_v, acc_sh.at[dst_v.at[j]], add=True)
            return 0
        lax.fori_loop(0, WPT, body, 0)

        plsc.subcore_barrier()
        pltpu.sync_copy(acc_sh.at[pl.ds(s * RPT, RPT)],
                        out_hbm.at[c, pl.ds(s * RPT, RPT)])

    return agg, deg


def _tc1_body(degp_ref, x_ref, w_ref, dinv_ref, hp_ref):
    degsum = 1.0 + degp_ref[0] + degp_ref[1]
    dinv = lax.rsqrt(degsum)
    dinv_ref[...] = dinv
    z = jnp.dot(x_ref[...], w_ref[...], preferred_element_type=jnp.float32)
    hp_ref[...] = z * dinv[:, 0:1]


def _tc1(degp, xp, W, interpret=False):
    return pl.pallas_call(
        _tc1_body,
        grid=(NB,),
        in_specs=[
            pl.BlockSpec((NC, BLK, 16), lambda i: (0, i, 0)),
            pl.BlockSpec((BLK, D), lambda i: (i, 0)),
            pl.BlockSpec((D, D), lambda i: (0, 0)),
        ],
        out_specs=[
            pl.BlockSpec((BLK, 16), lambda i: (i, 0)),
            pl.BlockSpec((BLK, D), lambda i: (i, 0)),
        ],
        out_shape=[
            jax.ShapeDtypeStruct((NP, 16), jnp.float32),
            jax.ShapeDtypeStruct((NP, D), jnp.float32),
        ],
        interpret=interpret,
    )(degp, xp, W)


def _tc2_body(s_ref, hp_ref, dinv_ref, b_ref, w_ref, out_ref):
    dinv = dinv_ref[...][:, 0:1]
    h = jnp.tanh((s_ref[0] + s_ref[1] + hp_ref[...]) * dinv + b_ref[...])
    z = jnp.dot(h, w_ref[...], preferred_element_type=jnp.float32)
    out_ref[...] = z * dinv


def _tc2(S, hp, dinv, b, W, interpret=False):
    return pl.pallas_call(
        _tc2_body,
        grid=(NB,),
        in_specs=[
            pl.BlockSpec((NC, BLK, D), lambda i: (0, i, 0)),
            pl.BlockSpec((BLK, D), lambda i: (i, 0)),
            pl.BlockSpec((BLK, 16), lambda i: (i, 0)),
            pl.BlockSpec((1, D), lambda i: (0, 0)),
            pl.BlockSpec((D, D), lambda i: (0, 0)),
        ],
        out_specs=pl.BlockSpec((BLK, D), lambda i: (i, 0)),
        out_shape=jax.ShapeDtypeStruct((NP, D), jnp.float32),
        interpret=interpret,
    )(S, hp, dinv, b, W)


def _tc4_body(s_ref, hp_ref, dinv_ref, b_ref, wl_ref, bl_ref, batch_ref,
              out_ref, acc_s, cnt_s):
    i = pl.program_id(0)
    dinv = dinv_ref[...][:, 0:1]
    h = jnp.tanh((s_ref[0] + s_ref[1] + hp_ref[...]) * dinv + b_ref[...])
    hf = jnp.tanh(jnp.dot(h, wl_ref[...], preferred_element_type=jnp.float32)
                  + bl_ref[...])
    bt = batch_ref[0]
    gi = lax.broadcasted_iota(jnp.int32, (G, BLK), 0)
    p = jnp.where(bt == gi, 1.0, 0.0)

    @pl.when(i == 0)
    def _():
        acc_s[...] = jnp.zeros_like(acc_s)
        cnt_s[...] = jnp.zeros_like(cnt_s)

    acc_s[...] += jnp.dot(p, hf, preferred_element_type=jnp.float32)
    cnt_s[...] += jnp.sum(p, axis=1, keepdims=True)

    @pl.when(i == NB - 1)
    def _():
        out_ref[...] = acc_s[...] / jnp.maximum(cnt_s[...], 1.0)


def _tc4(S, hp, dinv, b, Wlp, blp, batch3, interpret=False):
    return pl.pallas_call(
        _tc4_body,
        grid=(NB,),
        in_specs=[
            pl.BlockSpec((NC, BLK, D), lambda i: (0, i, 0)),
            pl.BlockSpec((BLK, D), lambda i: (i, 0)),
            pl.BlockSpec((BLK, 16), lambda i: (i, 0)),
            pl.BlockSpec((1, D), lambda i: (0, 0)),
            pl.BlockSpec((D, D), lambda i: (0, 0)),
            pl.BlockSpec((1, D), lambda i: (0, 0)),
            pl.BlockSpec((1, 1, BLK), lambda i: (i, 0, 0)),
        ],
        out_specs=pl.BlockSpec((G, D), lambda i: (0, 0)),
        out_shape=jax.ShapeDtypeStruct((G, D), jnp.float32),
        scratch_shapes=[
            pltpu.VMEM((G, D), jnp.float32),
            pltpu.VMEM((G, 1), jnp.float32),
        ],
        interpret=interpret,
    )(S, hp, dinv, b, Wlp, blp, batch3)


def kernel(x, edge_index, batch, W1, b1, W2, b2, W3, b3, Wl, bl):
    agg, deg = _sc_kernels()

    src = edge_index[0]
    dst = edge_index[1]
    pad = EPAD - E
    ar = jnp.arange(pad, dtype=jnp.int32)
    src_p = jnp.concatenate([src, ar % N]).reshape(NW, WPT, CW)
    dst_p = jnp.concatenate([dst, DUMMY0 + ar % NDUM]).reshape(NW, WPT, CW)

    xp = jnp.pad(x, ((0, NP - N), (0, 0)))
    batch3 = jnp.pad(batch, (0, NP - N), constant_values=G).reshape(NB, 1, BLK)
    Wlp = jnp.pad(Wl, ((0, 0), (0, D - Wl.shape[1])))
    blp = jnp.pad(bl, (0, D - bl.shape[0])).reshape(1, D)
    b1r = b1.reshape(1, D)
    b2r = b2.reshape(1, D)
    b3r = b3.reshape(1, D)

    degp = deg(dst_p)
    dinv, hp1 = _tc1(degp, xp, W1)
    S1 = agg(src_p, dst_p, hp1)
    hp2 = _tc2(S1, hp1, dinv, b1r, W2)
    S2 = agg(src_p, dst_p, hp2)
    hp3 = _tc2(S2, hp2, dinv, b2r, W3)
    S3 = agg(src_p, dst_p, hp3)
    pooled = _tc4(S3, hp3, dinv, b3r, Wlp, blp, batch3)
    return pooled[:, :Wl.shape[1]]


# speedup vs baseline: 24.3228x; 1.2931x over previous
import functools

import jax
import jax.numpy as jnp
from jax import lax
from jax.experimental import pallas as pl
from jax.experimental.pallas import tpu as pltpu
from jax.experimental.pallas import tpu_sc as plsc

N = 10000
E = 320000
D = 128
G = 64

NC = 2
NS = 16
NW = NC * NS
CW = 64
WPT = 160
CH = 16
NCH = WPT // CH
NBUF = 4
EPT = WPT * CW
EPAD = EPT * NW
NP = 10240
RPT = NP // NS
BLK = 512
NB = NP // BLK
DUMMY0 = N + 16
NDUM = NP - DUMMY0


@functools.lru_cache(maxsize=None)
def _sc_kernels():
    mesh = plsc.VectorSubcoreMesh(
        core_axis_name="c", subcore_axis_name="s",
        num_cores=NC, num_subcores=NS)

    @functools.partial(
        pl.kernel,
        out_type=jax.ShapeDtypeStruct((NC, NP, D), jnp.float32),
        mesh=mesh,
        scratch_types=[
            [pltpu.VMEM((CH, CW), jnp.int32) for _ in range(2)],
            [pltpu.VMEM((CH, CW), jnp.int32) for _ in range(2)],
            [pltpu.VMEM((CW, D), jnp.float32) for _ in range(NBUF)],
            pltpu.VMEM_SHARED((NP, D), jnp.float32),
            [pltpu.SemaphoreType.DMA for _ in range(NBUF)],
            [pltpu.SemaphoreType.DMA for _ in range(NBUF)],
            [pltpu.SemaphoreType.DMA for _ in range(2)],
        ],
    )
    def agg(src_hbm, dst_hbm, hp_hbm, out_hbm, sidx, didx, rows,
            acc_sh, gsem, ssem, isem):
        c = lax.axis_index("c")
        s = lax.axis_index("s")
        w = c * NS + s

        pltpu.sync_copy(src_hbm.at[w, 0], sidx[0])
        pltpu.sync_copy(dst_hbm.at[w, 0], didx[0])

        def zfill(i, _):
            rows[0][i // 8, pl.ds((i % 8) * 16, 16)] = jnp.zeros(
                (16,), jnp.float32)
            return 0
        lax.fori_loop(0, CW * 8, zfill, 0)

        def zcopy(i, _):
            pltpu.sync_copy(rows[0], acc_sh.at[pl.ds(s * RPT + i * CW, CW)])
            return 0
        lax.fori_loop(0, RPT // CW, zcopy, 0)

        for b in range(NBUF):
            pltpu.async_copy(hp_hbm.at[sidx[0].at[b]], rows[b], gsem[b])
        plsc.subcore_barrier()

        def chunk_body(ck, p):
            o = 1 - p
            for t in range(CH):
                j = ck * CH + t
                b = t % NBUF
                pltpu.make_async_copy(hp_hbm.at[sidx[p].at[t]], rows[b],
                                      gsem[b]).wait()
                pltpu.async_copy(rows[b], acc_sh.at[didx[p].at[t]], ssem[b],
                                 add=True)

                if t == 2:
                    @pl.when(ck + 1 < NCH)
                    def _():
                        pltpu.async_copy(src_hbm.at[w, ck + 1], sidx[o],
                                         isem[o])
                        pltpu.async_copy(dst_hbm.at[w, ck + 1], didx[o],
                                         isem[o])
                if t == CH - 2:
                    @pl.when(ck + 1 < NCH)
                    def _():
                        pltpu.make_async_copy(src_hbm.at[w, ck + 1], sidx[o],
                                              isem[o]).wait()
                        pltpu.make_async_copy(dst_hbm.at[w, ck + 1], didx[o],
                                              isem[o]).wait()

                b2 = (t + 2) % NBUF

                @pl.when(jnp.logical_and(j >= 2, j + 2 < WPT))
                def _():
                    pltpu.make_async_copy(rows[b2], acc_sh.at[didx[p].at[t]],
                                          ssem[b2]).wait()
                    if t + 2 < CH:
                        pltpu.async_copy(hp_hbm.at[sidx[p].at[t + 2]],
                                         rows[b2], gsem[b2])
                    else:
                        pltpu.async_copy(hp_hbm.at[sidx[o].at[t + 2 - CH]],
                                         rows[b2], gsem[b2])

        def body(cp, _):
            chunk_body(2 * cp, 0)
            chunk_body(2 * cp + 1, 1)
            return 0
        lax.fori_loop(0, NCH // 2, body, 0)

        for b in range(NBUF):
            pltpu.make_async_copy(rows[b], acc_sh.at[didx[1].at[b]],
                                  ssem[b]).wait()

        plsc.subcore_barrier()
        pltpu.sync_copy(acc_sh.at[pl.ds(s * RPT, RPT)],
                        out_hbm.at[c, pl.ds(s * RPT, RPT)])

    @functools.partial(
        pl.kernel,
        out_type=jax.ShapeDtypeStruct((NC, NP, 16), jnp.float32),
        mesh=mesh,
        scratch_types=[
            pltpu.VMEM((WPT, CW), jnp.int32),
            pltpu.VMEM((CW, 16), jnp.float32),
            pltpu.VMEM((CW, 16), jnp.float32),
            pltpu.VMEM_SHARED((NP, 16), jnp.float32),
        ],
    )
    def deg(dst_hbm, out_hbm, dst_v, ones_v, zb_v, acc_sh):
        c = lax.axis_index("c")
        s = lax.axis_index("s")
        w = c * NS + s

        def ofill(i, _):
            ones_v[i, :] = jnp.ones((16,), jnp.float32)
            zb_v[i, :] = jnp.zeros((16,), jnp.float32)
            return 0
        lax.fori_loop(0, CW, ofill, 0)

        def zcopy(i, _):
            pltpu.sync_copy(zb_v, acc_sh.at[pl.ds(s * RPT + i * CW, CW)])
            return 0
        lax.fori_loop(0, RPT // CW, zcopy, 0)

        pltpu.sync_copy(dst_hbm.at[w], dst_v)
        plsc.subcore_barrier()

        def body(j, _):
            pltpu.sync_copy(ones_v, acc_sh.at[dst_v.at[j]], add=True)
            return 0
        lax.fori_loop(0, WPT, body, 0)

        plsc.subcore_barrier()
        pltpu.sync_copy(acc_sh.at[pl.ds(s * RPT, RPT)],
                        out_hbm.at[c, pl.ds(s * RPT, RPT)])

    return agg, deg


def _tc1_body(degp_ref, x_ref, w_ref, dinv_ref, hp_ref):
    degsum = 1.0 + degp_ref[0] + degp_ref[1]
    dinv = lax.rsqrt(degsum)
    dinv_ref[...] = dinv
    z = jnp.dot(x_ref[...], w_ref[...], preferred_element_type=jnp.float32)
    hp_ref[...] = z * dinv[:, 0:1]


def _tc1(degp, xp, W, interpret=False):
    return pl.pallas_call(
        _tc1_body,
        grid=(NB,),
        in_specs=[
            pl.BlockSpec((NC, BLK, 16), lambda i: (0, i, 0)),
            pl.BlockSpec((BLK, D), lambda i: (i, 0)),
            pl.BlockSpec((D, D), lambda i: (0, 0)),
        ],
        out_specs=[
            pl.BlockSpec((BLK, 16), lambda i: (i, 0)),
            pl.BlockSpec((BLK, D), lambda i: (i, 0)),
        ],
        out_shape=[
            jax.ShapeDtypeStruct((NP, 16), jnp.float32),
            jax.ShapeDtypeStruct((NP, D), jnp.float32),
        ],
        interpret=interpret,
    )(degp, xp, W)


def _tc2_body(s_ref, hp_ref, dinv_ref, b_ref, w_ref, out_ref):
    dinv = dinv_ref[...][:, 0:1]
    h = jnp.tanh((s_ref[0] + s_ref[1] + hp_ref[...]) * dinv + b_ref[...])
    z = jnp.dot(h, w_ref[...], preferred_element_type=jnp.float32)
    out_ref[...] = z * dinv


def _tc2(S, hp, dinv, b, W, interpret=False):
    return pl.pallas_call(
        _tc2_body,
        grid=(NB,),
        in_specs=[
            pl.BlockSpec((NC, BLK, D), lambda i: (0, i, 0)),
            pl.BlockSpec((BLK, D), lambda i: (i, 0)),
            pl.BlockSpec((BLK, 16), lambda i: (i, 0)),
            pl.BlockSpec((1, D), lambda i: (0, 0)),
            pl.BlockSpec((D, D), lambda i: (0, 0)),
        ],
        out_specs=pl.BlockSpec((BLK, D), lambda i: (i, 0)),
        out_shape=jax.ShapeDtypeStruct((NP, D), jnp.float32),
        interpret=interpret,
    )(S, hp, dinv, b, W)


def _tc4_body(s_ref, hp_ref, dinv_ref, b_ref, wl_ref, bl_ref, batch_ref,
              out_ref, acc_s, cnt_s):
    i = pl.program_id(0)
    dinv = dinv_ref[...][:, 0:1]
    h = jnp.tanh((s_ref[0] + s_ref[1] + hp_ref[...]) * dinv + b_ref[...])
    hf = jnp.tanh(jnp.dot(h, wl_ref[...], preferred_element_type=jnp.float32)
                  + bl_ref[...])
    bt = batch_ref[0]
    gi = lax.broadcasted_iota(jnp.int32, (G, BLK), 0)
    p = jnp.where(bt == gi, 1.0, 0.0)

    @pl.when(i == 0)
    def _():
        acc_s[...] = jnp.zeros_like(acc_s)
        cnt_s[...] = jnp.zeros_like(cnt_s)

    acc_s[...] += jnp.dot(p, hf, preferred_element_type=jnp.float32)
    cnt_s[...] += jnp.sum(p, axis=1, keepdims=True)

    @pl.when(i == NB - 1)
    def _():
        out_ref[...] = acc_s[...] / jnp.maximum(cnt_s[...], 1.0)


def _tc4(S, hp, dinv, b, Wlp, blp, batch3, interpret=False):
    return pl.pallas_call(
        _tc4_body,
        grid=(NB,),
        in_specs=[
            pl.BlockSpec((NC, BLK, D), lambda i: (0, i, 0)),
            pl.BlockSpec((BLK, D), lambda i: (i, 0)),
            pl.BlockSpec((BLK, 16), lambda i: (i, 0)),
            pl.BlockSpec((1, D), lambda i: (0, 0)),
            pl.BlockSpec((D, D), lambda i: (0, 0)),
            pl.BlockSpec((1, D), lambda i: (0, 0)),
            pl.BlockSpec((1, 1, BLK), lambda i: (i, 0, 0)),
        ],
        out_specs=pl.BlockSpec((G, D), lambda i: (0, 0)),
        out_shape=jax.ShapeDtypeStruct((G, D), jnp.float32),
        scratch_shapes=[
            pltpu.VMEM((G, D), jnp.float32),
            pltpu.VMEM((G, 1), jnp.float32),
        ],
        interpret=interpret,
    )(S, hp, dinv, b, Wlp, blp, batch3)


def kernel(x, edge_index, batch, W1, b1, W2, b2, W3, b3, Wl, bl):
    agg, deg = _sc_kernels()

    src = edge_index[0]
    dst = edge_index[1]
    pad = EPAD - E
    ar = jnp.arange(pad, dtype=jnp.int32)
    src_p = jnp.concatenate([src, ar % N]).reshape(NW, NCH, CH, CW)
    dst_p = jnp.concatenate([dst, DUMMY0 + ar % NDUM]).reshape(NW, NCH, CH, CW)
    dst_p3 = dst_p.reshape(NW, WPT, CW)

    xp = jnp.pad(x, ((0, NP - N), (0, 0)))
    batch3 = jnp.pad(batch, (0, NP - N), constant_values=G).reshape(NB, 1, BLK)
    Wlp = jnp.pad(Wl, ((0, 0), (0, D - Wl.shape[1])))
    blp = jnp.pad(bl, (0, D - bl.shape[0])).reshape(1, D)
    b1r = b1.reshape(1, D)
    b2r = b2.reshape(1, D)
    b3r = b3.reshape(1, D)

    degp = deg(dst_p3)
    dinv, hp1 = _tc1(degp, xp, W1)
    S1 = agg(src_p, dst_p, hp1)
    hp2 = _tc2(S1, hp1, dinv, b1r, W2)
    S2 = agg(src_p, dst_p, hp2)
    hp3 = _tc2(S2, hp2, dinv, b2r, W3)
    S3 = agg(src_p, dst_p, hp3)
    pooled = _tc4(S3, hp3, dinv, b3r, Wlp, blp, batch3)
    return pooled[:, :Wl.shape[1]]


# speedup vs baseline: 24.6729x; 1.0144x over previous
import functools

import jax
import jax.numpy as jnp
from jax import lax
from jax.experimental import pallas as pl
from jax.experimental.pallas import tpu as pltpu
from jax.experimental.pallas import tpu_sc as plsc

N = 10000
E = 320000
D = 128
G = 64

NC = 2
NS = 16
NW = NC * NS
CW = 64
WPT = 160
CH = 16
NCH = WPT // CH
NBUF = 4
CW2 = 128
WPT2 = (WPT * CW) // CW2
EPT = WPT * CW
EPAD = EPT * NW
NP = 10240
RPT = NP // NS
BLK = 512
NB = NP // BLK
DUMMY0 = N + 16
NDUM = NP - DUMMY0


@functools.lru_cache(maxsize=None)
def _sc_kernels():
    mesh = plsc.VectorSubcoreMesh(
        core_axis_name="c", subcore_axis_name="s",
        num_cores=NC, num_subcores=NS)

    @functools.partial(
        pl.kernel,
        out_type=jax.ShapeDtypeStruct((NC, NP, D), jnp.float32),
        mesh=mesh,
        scratch_types=[
            [pltpu.VMEM((CH, CW), jnp.int32) for _ in range(2)],
            [pltpu.VMEM((CH, CW), jnp.int32) for _ in range(2)],
            [pltpu.VMEM((CW, D), jnp.float32) for _ in range(NBUF)],
            pltpu.VMEM_SHARED((NP, D), jnp.float32),
            [pltpu.SemaphoreType.DMA for _ in range(NBUF)],
            [pltpu.SemaphoreType.DMA for _ in range(NBUF)],
            [pltpu.SemaphoreType.DMA for _ in range(2)],
        ],
    )
    def agg(src_hbm, dst_hbm, hp_hbm, out_hbm, sidx, didx, rows,
            acc_sh, gsem, ssem, isem):
        c = lax.axis_index("c")
        s = lax.axis_index("s")
        w = c * NS + s

        pltpu.sync_copy(src_hbm.at[w, 0], sidx[0])
        pltpu.sync_copy(dst_hbm.at[w, 0], didx[0])

        def zfill(i, _):
            rows[0][i // 8, pl.ds((i % 8) * 16, 16)] = jnp.zeros(
                (16,), jnp.float32)
            return 0
        lax.fori_loop(0, CW * 8, zfill, 0)

        def zcopy(i, _):
            pltpu.sync_copy(rows[0], acc_sh.at[pl.ds(s * RPT + i * CW, CW)])
            return 0
        lax.fori_loop(0, RPT // CW, zcopy, 0)

        for b in range(NBUF):
            pltpu.async_copy(hp_hbm.at[sidx[0].at[b]], rows[b], gsem[b])
        plsc.subcore_barrier()

        def chunk_body(ck, p):
            o = 1 - p
            for t in range(CH):
                j = ck * CH + t
                b = t % NBUF
                pltpu.make_async_copy(hp_hbm.at[sidx[p].at[t]], rows[b],
                                      gsem[b]).wait()
                pltpu.async_copy(rows[b], acc_sh.at[didx[p].at[t]], ssem[b],
                                 add=True)

                if t == 2:
                    @pl.when(ck + 1 < NCH)
                    def _():
                        pltpu.async_copy(src_hbm.at[w, ck + 1], sidx[o],
                                         isem[o])
                        pltpu.async_copy(dst_hbm.at[w, ck + 1], didx[o],
                                         isem[o])
                if t == CH - 2:
                    @pl.when(ck + 1 < NCH)
                    def _():
                        pltpu.make_async_copy(src_hbm.at[w, ck + 1], sidx[o],
                                              isem[o]).wait()
                        pltpu.make_async_copy(dst_hbm.at[w, ck + 1], didx[o],
                                              isem[o]).wait()

                b2 = (t + 2) % NBUF

                @pl.when(jnp.logical_and(j >= 2, j + 2 < WPT))
                def _():
                    pltpu.make_async_copy(rows[b2], acc_sh.at[didx[p].at[t]],
                                          ssem[b2]).wait()
                    if t + 2 < CH:
                        pltpu.async_copy(hp_hbm.at[sidx[p].at[t + 2]],
                                         rows[b2], gsem[b2])
                    else:
                        pltpu.async_copy(hp_hbm.at[sidx[o].at[t + 2 - CH]],
                                         rows[b2], gsem[b2])

        def body(cp, _):
            chunk_body(2 * cp, 0)
            chunk_body(2 * cp + 1, 1)
            return 0
        lax.fori_loop(0, NCH // 2, body, 0)

        for b in range(NBUF):
            pltpu.make_async_copy(rows[b], acc_sh.at[didx[1].at[b]],
                                  ssem[b]).wait()

        plsc.subcore_barrier()
        pltpu.sync_copy(acc_sh.at[pl.ds(s * RPT, RPT)],
                        out_hbm.at[c, pl.ds(s * RPT, RPT)])

    @functools.partial(
        pl.kernel,
        out_type=jax.ShapeDtypeStruct((NC, NP, 16), jnp.float32),
        mesh=mesh,
        scratch_types=[
            pltpu.VMEM((WPT2, CW2), jnp.int32),
            pltpu.VMEM((CW2, 16), jnp.float32),
            pltpu.VMEM((CW2, 16), jnp.float32),
            pltpu.VMEM_SHARED((NP, 16), jnp.float32),
            [pltpu.SemaphoreType.DMA for _ in range(2)],
        ],
    )
    def deg(dst_hbm, out_hbm, dst_v, ones_v, zb_v, acc_sh, ssem):
        c = lax.axis_index("c")
        s = lax.axis_index("s")
        w = c * NS + s

        def ofill(i, _):
            ones_v[i, :] = jnp.ones((16,), jnp.float32)
            zb_v[i, :] = jnp.zeros((16,), jnp.float32)
            return 0
        lax.fori_loop(0, CW2, ofill, 0)

        def zcopy(i, _):
            pltpu.sync_copy(zb_v, acc_sh.at[pl.ds(s * RPT + i * CW2, CW2)])
            return 0
        lax.fori_loop(0, RPT // CW2, zcopy, 0)

        pltpu.sync_copy(dst_hbm.at[w], dst_v)
        plsc.subcore_barrier()

        def body(g, _):
            for q in range(2):
                j = 2 * g + q

                @pl.when(g > 0)
                def _():
                    pltpu.make_async_copy(ones_v, acc_sh.at[dst_v.at[j]],
                                          ssem[q]).wait()
                pltpu.async_copy(ones_v, acc_sh.at[dst_v.at[j]], ssem[q],
                                 add=True)
            return 0
        lax.fori_loop(0, WPT2 // 2, body, 0)
        for q in range(2):
            pltpu.make_async_copy(ones_v, acc_sh.at[dst_v.at[q]],
                                  ssem[q]).wait()

        plsc.subcore_barrier()
        pltpu.sync_copy(acc_sh.at[pl.ds(s * RPT, RPT)],
                        out_hbm.at[c, pl.ds(s * RPT, RPT)])

    return agg, deg


def _tc1a_body(x_ref, w_ref, z_ref):
    z_ref[...] = jnp.dot(x_ref[...], w_ref[...],
                         preferred_element_type=jnp.float32)


def _tc1a(xp, W, interpret=False):
    return pl.pallas_call(
        _tc1a_body,
        grid=(NB,),
        in_specs=[
            pl.BlockSpec((BLK, D), lambda i: (i, 0)),
            pl.BlockSpec((D, D), lambda i: (0, 0)),
        ],
        out_specs=pl.BlockSpec((BLK, D), lambda i: (i, 0)),
        out_shape=jax.ShapeDtypeStruct((NP, D), jnp.float32),
        interpret=interpret,
    )(xp, W)


def _tc1b_body(degp_ref, z_ref, dinv_ref, hp_ref):
    degsum = 1.0 + degp_ref[0] + degp_ref[1]
    dinv = lax.rsqrt(degsum)
    dinv_ref[...] = dinv
    hp_ref[...] = z_ref[...] * dinv[:, 0:1]


def _tc1b(degp, z, interpret=False):
    return pl.pallas_call(
        _tc1b_body,
        grid=(NB,),
        in_specs=[
            pl.BlockSpec((NC, BLK, 16), lambda i: (0, i, 0)),
            pl.BlockSpec((BLK, D), lambda i: (i, 0)),
        ],
        out_specs=[
            pl.BlockSpec((BLK, 16), lambda i: (i, 0)),
            pl.BlockSpec((BLK, D), lambda i: (i, 0)),
        ],
        out_shape=[
            jax.ShapeDtypeStruct((NP, 16), jnp.float32),
            jax.ShapeDtypeStruct((NP, D), jnp.float32),
        ],
        interpret=interpret,
    )(degp, z)


def _tc2_body(s_ref, hp_ref, dinv_ref, b_ref, w_ref, out_ref):
    dinv = dinv_ref[...][:, 0:1]
    h = jnp.tanh((s_ref[0] + s_ref[1] + hp_ref[...]) * dinv + b_ref[...])
    z = jnp.dot(h, w_ref[...], preferred_element_type=jnp.float32)
    out_ref[...] = z * dinv


def _tc2(S, hp, dinv, b, W, interpret=False):
    return pl.pallas_call(
        _tc2_body,
        grid=(NB,),
        in_specs=[
            pl.BlockSpec((NC, BLK, D), lambda i: (0, i, 0)),
            pl.BlockSpec((BLK, D), lambda i: (i, 0)),
            pl.BlockSpec((BLK, 16), lambda i: (i, 0)),
            pl.BlockSpec((1, D), lambda i: (0, 0)),
            pl.BlockSpec((D, D), lambda i: (0, 0)),
        ],
        out_specs=pl.BlockSpec((BLK, D), lambda i: (i, 0)),
        out_shape=jax.ShapeDtypeStruct((NP, D), jnp.float32),
        interpret=interpret,
    )(S, hp, dinv, b, W)


def _tc4_body(s_ref, hp_ref, dinv_ref, b_ref, wl_ref, bl_ref, batch_ref,
              out_ref, acc_s, cnt_s):
    i = pl.program_id(0)
    dinv = dinv_ref[...][:, 0:1]
    h = jnp.tanh((s_ref[0] + s_ref[1] + hp_ref[...]) * dinv + b_ref[...])
    hf = jnp.tanh(jnp.dot(h, wl_ref[...], preferred_element_type=jnp.float32)
                  + bl_ref[...])
    bt = batch_ref[0]
    gi = lax.broadcasted_iota(jnp.int32, (G, BLK), 0)
    p = jnp.where(bt == gi, 1.0, 0.0)

    @pl.when(i == 0)
    def _():
        acc_s[...] = jnp.zeros_like(acc_s)
        cnt_s[...] = jnp.zeros_like(cnt_s)

    acc_s[...] += jnp.dot(p, hf, preferred_element_type=jnp.float32)
    cnt_s[...] += jnp.sum(p, axis=1, keepdims=True)

    @pl.when(i == NB - 1)
    def _():
        out_ref[...] = acc_s[...] / jnp.maximum(cnt_s[...], 1.0)


def _tc4(S, hp, dinv, b, Wlp, blp, batch3, interpret=False):
    return pl.pallas_call(
        _tc4_body,
        grid=(NB,),
        in_specs=[
            pl.BlockSpec((NC, BLK, D), lambda i: (0, i, 0)),
            pl.BlockSpec((BLK, D), lambda i: (i, 0)),
            pl.BlockSpec((BLK, 16), lambda i: (i, 0)),
            pl.BlockSpec((1, D), lambda i: (0, 0)),
            pl.BlockSpec((D, D), lambda i: (0, 0)),
            pl.BlockSpec((1, D), lambda i: (0, 0)),
            pl.BlockSpec((1, 1, BLK), lambda i: (i, 0, 0)),
        ],
        out_specs=pl.BlockSpec((G, D), lambda i: (0, 0)),
        out_shape=jax.ShapeDtypeStruct((G, D), jnp.float32),
        scratch_shapes=[
            pltpu.VMEM((G, D), jnp.float32),
            pltpu.VMEM((G, 1), jnp.float32),
        ],
        interpret=interpret,
    )(S, hp, dinv, b, Wlp, blp, batch3)


def kernel(x, edge_index, batch, W1, b1, W2, b2, W3, b3, Wl, bl):
    agg, deg = _sc_kernels()

    src = edge_index[0]
    dst = edge_index[1]
    pad = EPAD - E
    ar = jnp.arange(pad, dtype=jnp.int32)
    src_p = jnp.concatenate([src, ar % N]).reshape(NW, NCH, CH, CW)
    dst_p = jnp.concatenate([dst, DUMMY0 + ar % NDUM]).reshape(NW, NCH, CH, CW)
    dst_p2 = dst_p.reshape(NW, WPT2, CW2)

    xp = jnp.pad(x, ((0, NP - N), (0, 0)))
    batch3 = jnp.pad(batch, (0, NP - N), constant_values=G).reshape(NB, 1, BLK)
    Wlp = jnp.pad(Wl, ((0, 0), (0, D - Wl.shape[1])))
    blp = jnp.pad(bl, (0, D - bl.shape[0])).reshape(1, D)
    b1r = b1.reshape(1, D)
    b2r = b2.reshape(1, D)
    b3r = b3.reshape(1, D)

    degp = deg(dst_p2)
    z1 = _tc1a(xp, W1)
    dinv, hp1 = _tc1b(degp, z1)
    S1 = agg(src_p, dst_p, hp1)
    hp2 = _tc2(S1, hp1, dinv, b1r, W2)
    S2 = agg(src_p, dst_p, hp2)
    hp3 = _tc2(S2, hp2, dinv, b2r, W3)
    S3 = agg(src_p, dst_p, hp3)
    pooled = _tc4(S3, hp3, dinv, b3r, Wlp, blp, batch3)
    return pooled[:, :Wl.shape[1]]


# speedup vs baseline: 24.7158x; 1.0017x over previous
import functools

import jax
import jax.numpy as jnp
from jax import lax
from jax.experimental import pallas as pl
from jax.experimental.pallas import tpu as pltpu
from jax.experimental.pallas import tpu_sc as plsc

N = 10000
E = 320000
D = 128
G = 64

NC = 2
NS = 16
NW = NC * NS
CW = 64
WPT = 160
CH = 16
NCH = WPT // CH
NBUF = 4
CW2 = 128
WPT2 = (WPT * CW) // CW2
EPT = WPT * CW
EPAD = EPT * NW
NP = 10240
RPT = NP // NS
BLK = 512
NB = NP // BLK
DUMMY0 = N + 16
NDUM = NP - DUMMY0


@functools.lru_cache(maxsize=None)
def _sc_kernels():
    mesh = plsc.VectorSubcoreMesh(
        core_axis_name="c", subcore_axis_name="s",
        num_cores=NC, num_subcores=NS)

    @functools.partial(
        pl.kernel,
        out_type=jax.ShapeDtypeStruct((NC, NP, D), jnp.float32),
        mesh=mesh,
        scratch_types=[
            [pltpu.VMEM((CH, CW), jnp.int32) for _ in range(2)],
            [pltpu.VMEM((CH, CW), jnp.int32) for _ in range(2)],
            [pltpu.VMEM((CW, D), jnp.float32) for _ in range(NBUF)],
            pltpu.VMEM_SHARED((NP, D), jnp.float32),
            [pltpu.SemaphoreType.DMA for _ in range(NBUF)],
            [pltpu.SemaphoreType.DMA for _ in range(NBUF)],
            [pltpu.SemaphoreType.DMA for _ in range(2)],
        ],
    )
    def agg(src_hbm, dst_hbm, hp_hbm, out_hbm, sidx, didx, rows,
            acc_sh, gsem, ssem, isem):
        c = lax.axis_index("c")
        s = lax.axis_index("s")
        w = c * NS + s

        pltpu.sync_copy(src_hbm.at[w, 0], sidx[0])
        pltpu.sync_copy(dst_hbm.at[w, 0], didx[0])

        def zfill(i, _):
            rows[0][i // 8, pl.ds((i % 8) * 16, 16)] = jnp.zeros(
                (16,), jnp.float32)
            return 0
        lax.fori_loop(0, CW * 8, zfill, 0)

        def zcopy(i, _):
            pltpu.async_copy(rows[0], acc_sh.at[pl.ds(s * RPT + i * CW, CW)],
                             isem[0])
            return 0
        lax.fori_loop(0, RPT // CW, zcopy, 0)

        def zdrain(i, _):
            pltpu.make_async_copy(
                rows[0], acc_sh.at[pl.ds(s * RPT + i * CW, CW)],
                isem[0]).wait()
            return 0
        lax.fori_loop(0, RPT // CW, zdrain, 0)

        for b in range(NBUF):
            pltpu.async_copy(hp_hbm.at[sidx[0].at[b]], rows[b], gsem[b])
        plsc.subcore_barrier()

        def chunk_body(ck, p):
            o = 1 - p
            for t in range(CH):
                j = ck * CH + t
                b = t % NBUF
                pltpu.make_async_copy(hp_hbm.at[sidx[p].at[t]], rows[b],
                                      gsem[b]).wait()
                pltpu.async_copy(rows[b], acc_sh.at[didx[p].at[t]], ssem[b],
                                 add=True)

                if t == 2:
                    @pl.when(ck + 1 < NCH)
                    def _():
                        pltpu.async_copy(src_hbm.at[w, ck + 1], sidx[o],
                                         isem[o])
                        pltpu.async_copy(dst_hbm.at[w, ck + 1], didx[o],
                                         isem[o])
                if t == CH - 2:
                    @pl.when(ck + 1 < NCH)
                    def _():
                        pltpu.make_async_copy(src_hbm.at[w, ck + 1], sidx[o],
                                              isem[o]).wait()
                        pltpu.make_async_copy(dst_hbm.at[w, ck + 1], didx[o],
                                              isem[o]).wait()

                b2 = (t + 2) % NBUF

                @pl.when(jnp.logical_and(j >= 2, j + 2 < WPT))
                def _():
                    pltpu.make_async_copy(rows[b2], acc_sh.at[didx[p].at[t]],
                                          ssem[b2]).wait()
                    if t + 2 < CH:
                        pltpu.async_copy(hp_hbm.at[sidx[p].at[t + 2]],
                                         rows[b2], gsem[b2])
                    else:
                        pltpu.async_copy(hp_hbm.at[sidx[o].at[t + 2 - CH]],
                                         rows[b2], gsem[b2])

        def body(cp, _):
            chunk_body(2 * cp, 0)
            chunk_body(2 * cp + 1, 1)
            return 0
        lax.fori_loop(0, NCH // 2, body, 0)

        for b in range(NBUF):
            pltpu.make_async_copy(rows[b], acc_sh.at[didx[1].at[b]],
                                  ssem[b]).wait()

        plsc.subcore_barrier()
        pltpu.sync_copy(acc_sh.at[pl.ds(s * RPT, RPT)],
                        out_hbm.at[c, pl.ds(s * RPT, RPT)])

    @functools.partial(
        pl.kernel,
        out_type=jax.ShapeDtypeStruct((NC, NP, 16), jnp.float32),
        mesh=mesh,
        scratch_types=[
            pltpu.VMEM((WPT2, CW2), jnp.int32),
            pltpu.VMEM((CW2, 16), jnp.float32),
            pltpu.VMEM((CW2, 16), jnp.float32),
            pltpu.VMEM_SHARED((NP, 16), jnp.float32),
            [pltpu.SemaphoreType.DMA for _ in range(2)],
        ],
    )
    def deg(dst_hbm, out_hbm, dst_v, ones_v, zb_v, acc_sh, ssem):
        c = lax.axis_index("c")
        s = lax.axis_index("s")
        w = c * NS + s

        def ofill(i, _):
            ones_v[i, :] = jnp.ones((16,), jnp.float32)
            zb_v[i, :] = jnp.zeros((16,), jnp.float32)
            return 0
        lax.fori_loop(0, CW2, ofill, 0)

        def zcopy(i, _):
            pltpu.sync_copy(zb_v, acc_sh.at[pl.ds(s * RPT + i * CW2, CW2)])
            return 0
        lax.fori_loop(0, RPT // CW2, zcopy, 0)

        pltpu.sync_copy(dst_hbm.at[w], dst_v)
        plsc.subcore_barrier()

        def body(g, _):
            for q in range(2):
                j = 2 * g + q

                @pl.when(g > 0)
                def _():
                    pltpu.make_async_copy(ones_v, acc_sh.at[dst_v.at[j]],
                                          ssem[q]).wait()
                pltpu.async_copy(ones_v, acc_sh.at[dst_v.at[j]], ssem[q],
                                 add=True)
            return 0
        lax.fori_loop(0, WPT2 // 2, body, 0)
        for q in range(2):
            pltpu.make_async_copy(ones_v, acc_sh.at[dst_v.at[q]],
                                  ssem[q]).wait()

        plsc.subcore_barrier()
        pltpu.sync_copy(acc_sh.at[pl.ds(s * RPT, RPT)],
                        out_hbm.at[c, pl.ds(s * RPT, RPT)])

    return agg, deg


def _tc1a_body(x_ref, w_ref, z_ref):
    z_ref[...] = jnp.dot(x_ref[...], w_ref[...],
                         preferred_element_type=jnp.float32)


def _tc1a(xp, W, interpret=False):
    return pl.pallas_call(
        _tc1a_body,
        grid=(NB,),
        in_specs=[
            pl.BlockSpec((BLK, D), lambda i: (i, 0)),
            pl.BlockSpec((D, D), lambda i: (0, 0)),
        ],
        out_specs=pl.BlockSpec((BLK, D), lambda i: (i, 0)),
        out_shape=jax.ShapeDtypeStruct((NP, D), jnp.float32),
        interpret=interpret,
    )(xp, W)


def _tc1b_body(degp_ref, z_ref, dinv_ref, hp_ref):
    degsum = 1.0 + degp_ref[0] + degp_ref[1]
    dinv = lax.rsqrt(degsum)
    dinv_ref[...] = dinv
    hp_ref[...] = z_ref[...] * dinv[:, 0:1]


def _tc1b(degp, z, interpret=False):
    return pl.pallas_call(
        _tc1b_body,
        grid=(NB,),
        in_specs=[
            pl.BlockSpec((NC, BLK, 16), lambda i: (0, i, 0)),
            pl.BlockSpec((BLK, D), lambda i: (i, 0)),
        ],
        out_specs=[
            pl.BlockSpec((BLK, 16), lambda i: (i, 0)),
            pl.BlockSpec((BLK, D), lambda i: (i, 0)),
        ],
        out_shape=[
            jax.ShapeDtypeStruct((NP, 16), jnp.float32),
            jax.ShapeDtypeStruct((NP, D), jnp.float32),
        ],
        interpret=interpret,
    )(degp, z)


def _tc2_body(s_ref, hp_ref, dinv_ref, b_ref, w_ref, out_ref):
    dinv = dinv_ref[...][:, 0:1]
    h = jnp.tanh((s_ref[0] + s_ref[1] + hp_ref[...]) * dinv + b_ref[...])
    z = jnp.dot(h, w_ref[...], preferred_element_type=jnp.float32)
    out_ref[...] = z * dinv


def _tc2(S, hp, dinv, b, W, interpret=False):
    return pl.pallas_call(
        _tc2_body,
        grid=(NB,),
        in_specs=[
            pl.BlockSpec((NC, BLK, D), lambda i: (0, i, 0)),
            pl.BlockSpec((BLK, D), lambda i: (i, 0)),
            pl.BlockSpec((BLK, 16), lambda i: (i, 0)),
            pl.BlockSpec((1, D), lambda i: (0, 0)),
            pl.BlockSpec((D, D), lambda i: (0, 0)),
        ],
        out_specs=pl.BlockSpec((BLK, D), lambda i: (i, 0)),
        out_shape=jax.ShapeDtypeStruct((NP, D), jnp.float32),
        interpret=interpret,
    )(S, hp, dinv, b, W)


def _tc4_body(s_ref, hp_ref, dinv_ref, b_ref, wl_ref, bl_ref, batch_ref,
              out_ref, acc_s, cnt_s):
    i = pl.program_id(0)
    dinv = dinv_ref[...][:, 0:1]
    h = jnp.tanh((s_ref[0] + s_ref[1] + hp_ref[...]) * dinv + b_ref[...])
    hf = jnp.tanh(jnp.dot(h, wl_ref[...], preferred_element_type=jnp.float32)
                  + bl_ref[...])
    bt = batch_ref[0]
    gi = lax.broadcasted_iota(jnp.int32, (G, BLK), 0)
    p = jnp.where(bt == gi, 1.0, 0.0)

    @pl.when(i == 0)
    def _():
        acc_s[...] = jnp.zeros_like(acc_s)
        cnt_s[...] = jnp.zeros_like(cnt_s)

    acc_s[...] += jnp.dot(p, hf, preferred_element_type=jnp.float32)
    cnt_s[...] += jnp.sum(p, axis=1, keepdims=True)

    @pl.when(i == NB - 1)
    def _():
        out_ref[...] = acc_s[...] / jnp.maximum(cnt_s[...], 1.0)


def _tc4(S, hp, dinv, b, Wlp, blp, batch3, interpret=False):
    return pl.pallas_call(
        _tc4_body,
        grid=(NB,),
        in_specs=[
            pl.BlockSpec((NC, BLK, D), lambda i: (0, i, 0)),
            pl.BlockSpec((BLK, D), lambda i: (i, 0)),
            pl.BlockSpec((BLK, 16), lambda i: (i, 0)),
            pl.BlockSpec((1, D), lambda i: (0, 0)),
            pl.BlockSpec((D, D), lambda i: (0, 0)),
            pl.BlockSpec((1, D), lambda i: (0, 0)),
            pl.BlockSpec((1, 1, BLK), lambda i: (i, 0, 0)),
        ],
        out_specs=pl.BlockSpec((G, D), lambda i: (0, 0)),
        out_shape=jax.ShapeDtypeStruct((G, D), jnp.float32),
        scratch_shapes=[
            pltpu.VMEM((G, D), jnp.float32),
            pltpu.VMEM((G, 1), jnp.float32),
        ],
        interpret=interpret,
    )(S, hp, dinv, b, Wlp, blp, batch3)


def kernel(x, edge_index, batch, W1, b1, W2, b2, W3, b3, Wl, bl):
    agg, deg = _sc_kernels()

    src = edge_index[0]
    dst = edge_index[1]
    pad = EPAD - E
    ar = jnp.arange(pad, dtype=jnp.int32)
    src_p = jnp.concatenate([src, ar % N]).reshape(NW, NCH, CH, CW)
    dst_p = jnp.concatenate([dst, DUMMY0 + ar % NDUM]).reshape(NW, NCH, CH, CW)
    dst_p2 = dst_p.reshape(NW, WPT2, CW2)

    xp = jnp.pad(x, ((0, NP - N), (0, 0)))
    batch3 = jnp.pad(batch, (0, NP - N), constant_values=G).reshape(NB, 1, BLK)
    Wlp = jnp.pad(Wl, ((0, 0), (0, D - Wl.shape[1])))
    blp = jnp.pad(bl, (0, D - bl.shape[0])).reshape(1, D)
    b1r = b1.reshape(1, D)
    b2r = b2.reshape(1, D)
    b3r = b3.reshape(1, D)

    degp = deg(dst_p2)
    z1 = _tc1a(xp, W1)
    dinv, hp1 = _tc1b(degp, z1)
    S1 = agg(src_p, dst_p, hp1)
    hp2 = _tc2(S1, hp1, dinv, b1r, W2)
    S2 = agg(src_p, dst_p, hp2)
    hp3 = _tc2(S2, hp2, dinv, b2r, W3)
    S3 = agg(src_p, dst_p, hp3)
    pooled = _tc4(S3, hp3, dinv, b3r, Wlp, blp, batch3)
    return pooled[:, :Wl.shape[1]]
